# baseline (speedup 1.0000x reference)
import functools
import math

import jax
import jax.numpy as jnp
from jax import lax
from jax.experimental import pallas as pl
from jax.experimental.pallas import tpu as pltpu
from jax.experimental.pallas import tpu_sc as plsc

N = 10000
E = 320000
D = 128
C = 10
G = 64
NP = 10240

NC = 2
NS = 16
NW = NC * NS
EPT = E // NW
CH = 250
NCHUNK = EPT // CH
RPT = NP // NS

BLK = 2560
NBLK = NP // BLK


CPS = 4
NOUTER = NCHUNK // (2 * CPS)


def _sc_agg_body(x_hbm, srcdst_hbm, zeros_hbm, out_hbm,
                 acc_sp, x_sp, isA, idA, isB, idB, rows0, rows1,
                 gsem0, gsem1, ssem0, ssem1, isemA, isemB):
    rows = (rows0, rows1)
    gsem = (gsem0, gsem1)
    ssem = (ssem0, ssem1)
    c = lax.axis_index("c")
    s = lax.axis_index("s")
    w = c * NS + s

    def idx_load(iS, iD, base, sem):
        pltpu.async_copy(srcdst_hbm.at[0, w, pl.ds(base, CPS)], iS, sem)
        pltpu.async_copy(srcdst_hbm.at[1, w, pl.ds(base, CPS)], iD, sem)

    def idx_wait(iS, iD, base, sem):
        pltpu.make_async_copy(
            srcdst_hbm.at[0, w, pl.ds(base, CPS)], iS, sem).wait()
        pltpu.make_async_copy(
            srcdst_hbm.at[1, w, pl.ds(base, CPS)], iD, sem).wait()

    idx_load(isA, idA, 0, isemA)
    pltpu.sync_copy(x_hbm.at[pl.ds(s * RPT, RPT)],
                    x_sp.at[pl.ds(s * RPT, RPT)])

    @pl.when(c == 0)
    def _():
        pltpu.sync_copy(x_hbm.at[pl.ds(s * RPT, RPT)],
                        acc_sp.at[pl.ds(s * RPT, RPT)])

    @pl.when(c != 0)
    def _():
        pltpu.sync_copy(zeros_hbm, acc_sp.at[pl.ds(s * RPT, RPT)])

    plsc.subcore_barrier()
    idx_wait(isA, idA, 0, isemA)
    pltpu.async_copy(x_sp.at[isA.at[0]], rows0, gsem0)

    def substage(t, iS, iD, nS, nD, sem_n, first, is_b):
        for k in range(CPS):
            b = k % 2
            o = 1 - b
            if k == 0:
                if first:
                    idx_load(nS, nD, t * 2 * CPS + CPS, sem_n)
                else:
                    pltpu.make_async_copy(
                        rows[o], acc_sp.at[iD.at[0]], ssem[o]).wait()
                    if is_b:
                        @pl.when(t + 1 < NOUTER)
                        def _():
                            idx_load(nS, nD, (t + 1) * 2 * CPS, sem_n)
                    else:
                        idx_load(nS, nD, t * 2 * CPS + CPS, sem_n)
            else:
                pltpu.make_async_copy(
                    rows[o], acc_sp.at[iD.at[k - 1]], ssem[o]).wait()

            if k + 1 < CPS:
                pltpu.async_copy(x_sp.at[iS.at[k + 1]], rows[o], gsem[o])
            else:
                if is_b:
                    @pl.when(t + 1 < NOUTER)
                    def _():
                        idx_wait(nS, nD, (t + 1) * 2 * CPS, sem_n)
                        pltpu.async_copy(x_sp.at[nS.at[0]], rows[o], gsem[o])
                else:
                    idx_wait(nS, nD, t * 2 * CPS + CPS, sem_n)
                    pltpu.async_copy(x_sp.at[nS.at[0]], rows[o], gsem[o])

            pltpu.make_async_copy(x_sp.at[iS.at[k]], rows[b], gsem[b]).wait()
            pltpu.async_copy(rows[b], acc_sp.at[iD.at[k]], ssem[b], add=True)

    def outer(t, carry):
        @pl.when(t == 0)
        def _():
            substage(t, isA, idA, isB, idB, isemB, True, False)

        @pl.when(t != 0)
        def _():
            substage(t, isA, idA, isB, idB, isemB, False, False)

        substage(t, isB, idB, isA, idA, isemA, False, True)
        return carry

    lax.fori_loop(0, NOUTER, outer, 0)

    pltpu.make_async_copy(
        rows[(CPS - 1) % 2], acc_sp.at[idB.at[CPS - 1]],
        ssem[(CPS - 1) % 2]).wait()
    plsc.subcore_barrier()
    pltpu.sync_copy(acc_sp.at[pl.ds(s * RPT, RPT)],
                    out_hbm.at[c].at[pl.ds(s * RPT, RPT)])


@jax.jit
def _sc_agg(x, srcdst, zeros_blk):
    return pl.kernel(
        _sc_agg_body,
        out_type=jax.ShapeDtypeStruct((NC, NP, D), jnp.bfloat16),
        mesh=plsc.VectorSubcoreMesh(core_axis_name="c", subcore_axis_name="s"),
        compiler_params=pltpu.CompilerParams(use_tc_tiling_on_sc=False),
        scratch_types=[
            pltpu.MemorySpace.VMEM_SHARED((NP, D), jnp.bfloat16),
            pltpu.MemorySpace.VMEM_SHARED((NP, D), jnp.bfloat16),
            pltpu.MemorySpace.VMEM((CPS, CH), jnp.int32),
            pltpu.MemorySpace.VMEM((CPS, CH), jnp.int32),
            pltpu.MemorySpace.VMEM((CPS, CH), jnp.int32),
            pltpu.MemorySpace.VMEM((CPS, CH), jnp.int32),
            pltpu.MemorySpace.VMEM((CH, D), jnp.bfloat16),
            pltpu.MemorySpace.VMEM((CH, D), jnp.bfloat16),
        ] + [pltpu.SemaphoreType.DMA] * 6,
    )(x, srcdst, zeros_blk)


def _mlp_pool_body(p_ref, seg_ref, w1_ref, b1_ref, w2_ref, b2_ref,
                   sc_ref, sb_ref, y_ref, pool_ref, pacc):
    i = pl.program_id(0)
    h = p_ref[0].astype(jnp.float32) + p_ref[1].astype(jnp.float32)
    h1 = jnp.maximum(
        jnp.dot(h, w1_ref[...], preferred_element_type=jnp.float32)
        + b1_ref[...], 0.0)
    h2 = jnp.maximum(
        jnp.dot(h1, w2_ref[...], preferred_element_type=jnp.float32)
        + b2_ref[...], 0.0)
    y = h2 * sc_ref[...] + sb_ref[...]
    y_ref[...] = y.astype(jnp.bfloat16)

    oh = (lax.broadcasted_iota(jnp.int32, (G, BLK), 0)
          == seg_ref[0]).astype(jnp.float32)

    @pl.when(i == 0)
    def _():
        pacc[...] = jnp.zeros_like(pacc)

    pacc[...] += jnp.dot(oh, y, preferred_element_type=jnp.float32)

    @pl.when(i == pl.num_programs(0) - 1)
    def _():
        pool_ref[...] = pacc[...]


@jax.jit
def _mlp_pool(p, seg, w1, b1, w2, b2, scale, bias):
    return pl.pallas_call(
        _mlp_pool_body,
        grid=(NBLK,),
        in_specs=[
            pl.BlockSpec((NC, BLK, D), lambda i: (0, i, 0)),
            pl.BlockSpec((1, 1, BLK), lambda i: (i, 0, 0)),
            pl.BlockSpec((D, D), lambda i: (0, 0)),
            pl.BlockSpec((1, D), lambda i: (0, 0)),
            pl.BlockSpec((D, D), lambda i: (0, 0)),
            pl.BlockSpec((1, D), lambda i: (0, 0)),
            pl.BlockSpec((1, D), lambda i: (0, 0)),
            pl.BlockSpec((1, D), lambda i: (0, 0)),
        ],
        out_specs=[
            pl.BlockSpec((BLK, D), lambda i: (i, 0)),
            pl.BlockSpec((G, D), lambda i: (0, 0)),
        ],
        out_shape=[
            jax.ShapeDtypeStruct((NP, D), jnp.bfloat16),
            jax.ShapeDtypeStruct((G, D), jnp.float32),
        ],
        scratch_shapes=[pltpu.VMEM((G, D), jnp.float32)],
    )(p, seg, w1, b1, w2, b2, scale, bias)


def _pool_body(x_ref, seg_ref, pool_ref, pacc):
    i = pl.program_id(0)
    oh = (lax.broadcasted_iota(jnp.int32, (G, BLK), 0)
          == seg_ref[0]).astype(jnp.float32)

    @pl.when(i == 0)
    def _():
        pacc[...] = jnp.zeros_like(pacc)

    pacc[...] += jnp.dot(oh, x_ref[...], preferred_element_type=jnp.float32)

    @pl.when(i == pl.num_programs(0) - 1)
    def _():
        pool_ref[...] = pacc[...]


@jax.jit
def _pool(x, seg):
    return pl.pallas_call(
        _pool_body,
        grid=(NBLK,),
        in_specs=[
            pl.BlockSpec((BLK, D), lambda i: (i, 0)),
            pl.BlockSpec((1, 1, BLK), lambda i: (i, 0, 0)),
        ],
        out_specs=pl.BlockSpec((G, D), lambda i: (0, 0)),
        out_shape=jax.ShapeDtypeStruct((G, D), jnp.float32),
        scratch_shapes=[pltpu.VMEM((G, D), jnp.float32)],
    )(x, seg)


def _head_body(pools_ref, fc1w_ref, fc1b_ref, fc1s_ref, fc1t_ref,
               fc2w_ref, fc2b_ref, fc2s_ref, fc2t_ref,
               linw_ref, linb_ref, out_ref):
    def fc(h, w, b, s, t):
        z = jnp.maximum(
            jnp.dot(h, w[...], preferred_element_type=jnp.float32) + b[...],
            0.0)
        return z * s[...] + t[...]

    g = fc(pools_ref[0], fc1w_ref, fc1b_ref, fc1s_ref, fc1t_ref)
    acc = g
    for i in range(1, 6):
        g = fc(g + pools_ref[i], fc2w_ref, fc2b_ref, fc2s_ref, fc2t_ref)
        acc = acc + g
    logits = (jnp.dot(acc, linw_ref[...], preferred_element_type=jnp.float32)
              + linb_ref[...])
    m = jnp.max(logits, axis=-1, keepdims=True)
    z = logits - m
    out_ref[...] = z - jnp.log(jnp.sum(jnp.exp(z), axis=-1, keepdims=True))


@jax.jit
def _head(pools, fc1w, fc1b, fc1s, fc1t, fc2w, fc2b, fc2s, fc2t, linw, linb):
    return pl.pallas_call(
        _head_body,
        out_shape=jax.ShapeDtypeStruct((G, C), jnp.float32),
    )(pools, fc1w, fc1b, fc1s, fc1t, fc2w, fc2b, fc2s, fc2t, linw, linb)


_BN = 1.0 / math.sqrt(1.0 + 1e-5)


def kernel(x, edge_index, batch, params):
    srcdst = edge_index.astype(jnp.int32).reshape(2, NW, NCHUNK, CH)
    seg = jnp.pad(batch.astype(jnp.int32), (0, NP - N),
                  constant_values=G).reshape(NBLK, 1, BLK)
    zeros_blk = jnp.zeros((RPT, D), jnp.bfloat16)
    x = jnp.pad(x, ((0, NP - N), (0, 0)))
    p = params

    def row(v):
        return v.reshape(1, -1)

    pools = [_pool(x, seg)]
    h = x.astype(jnp.bfloat16)
    for c in ["c1", "c2", "c3", "c4", "c5"]:
        parts = _sc_agg(h, srcdst, zeros_blk)
        h, pl_c = _mlp_pool(parts, seg,
                            p[c + "_W1"], row(p[c + "_b1"]),
                            p[c + "_W2"], row(p[c + "_b2"]),
                            row(p[c + "_g"] * _BN), row(p[c + "_bb"]))
        pools.append(pl_c)

    pools = jnp.stack(pools)
    return _head(pools,
                 p["fc1_W"], row(p["fc1_b"]), row(p["fc1_g"] * _BN),
                 row(p["fc1_bb"]),
                 p["fc2_W"], row(p["fc2_b"]), row(p["fc2_g"] * _BN),
                 row(p["fc2_bb"]),
                 p["lin_W"], row(p["lin_b"]))


# speedup vs baseline: 1.2328x; 1.2328x over previous
import functools
import math

import jax
import jax.numpy as jnp
from jax import lax
from jax.experimental import pallas as pl
from jax.experimental.pallas import tpu as pltpu
from jax.experimental.pallas import tpu_sc as plsc

N = 10000
E = 320000
D = 128
C = 10
G = 64
NP = 10240

NC = 2
NS = 16
NW = NC * NS
EPT = E // NW
CH = 500
NCHUNK = EPT // CH
RPT = NP // NS

BLK = 2560
NBLK = NP // BLK


CPS = 2
NOUTER = NCHUNK // (2 * CPS)


def _sc_agg_body(x_hbm, srcdst_hbm, zeros_hbm, out_hbm,
                 acc_sp, isA, idA, isB, idB, rows0, rows1,
                 gsem0, gsem1, ssem0, ssem1, isemA, isemB):
    rows = (rows0, rows1)
    gsem = (gsem0, gsem1)
    ssem = (ssem0, ssem1)
    c = lax.axis_index("c")
    s = lax.axis_index("s")
    w = c * NS + s

    def idx_load(iS, iD, base, sem):
        pltpu.async_copy(srcdst_hbm.at[0, w, pl.ds(base, CPS)], iS, sem)
        pltpu.async_copy(srcdst_hbm.at[1, w, pl.ds(base, CPS)], iD, sem)

    def idx_wait(iS, iD, base, sem):
        pltpu.make_async_copy(
            srcdst_hbm.at[0, w, pl.ds(base, CPS)], iS, sem).wait()
        pltpu.make_async_copy(
            srcdst_hbm.at[1, w, pl.ds(base, CPS)], iD, sem).wait()

    idx_load(isA, idA, 0, isemA)
    idx_wait(isA, idA, 0, isemA)
    pltpu.async_copy(x_hbm.at[isA.at[0]], rows0, gsem0)

    @pl.when(c == 0)
    def _():
        pltpu.sync_copy(x_hbm.at[pl.ds(s * RPT, RPT)],
                        acc_sp.at[pl.ds(s * RPT, RPT)])

    @pl.when(c != 0)
    def _():
        pltpu.sync_copy(zeros_hbm, acc_sp.at[pl.ds(s * RPT, RPT)])

    plsc.subcore_barrier()

    def substage(t, iS, iD, nS, nD, sem_n, first, is_b):
        for k in range(CPS):
            b = k % 2
            o = 1 - b
            if k == 0:
                if first:
                    idx_load(nS, nD, t * 2 * CPS + CPS, sem_n)
                else:
                    pltpu.make_async_copy(
                        rows[o], acc_sp.at[iD.at[0]], ssem[o]).wait()
                    if is_b:
                        @pl.when(t + 1 < NOUTER)
                        def _():
                            idx_load(nS, nD, (t + 1) * 2 * CPS, sem_n)
                    else:
                        idx_load(nS, nD, t * 2 * CPS + CPS, sem_n)
            else:
                pltpu.make_async_copy(
                    rows[o], acc_sp.at[iD.at[k - 1]], ssem[o]).wait()

            if k + 1 < CPS:
                pltpu.async_copy(x_hbm.at[iS.at[k + 1]], rows[o], gsem[o])
            else:
                if is_b:
                    @pl.when(t + 1 < NOUTER)
                    def _():
                        idx_wait(nS, nD, (t + 1) * 2 * CPS, sem_n)
                        pltpu.async_copy(x_hbm.at[nS.at[0]], rows[o], gsem[o])
                else:
                    idx_wait(nS, nD, t * 2 * CPS + CPS, sem_n)
                    pltpu.async_copy(x_hbm.at[nS.at[0]], rows[o], gsem[o])

            pltpu.make_async_copy(x_hbm.at[iS.at[k]], rows[b], gsem[b]).wait()
            pltpu.async_copy(rows[b], acc_sp.at[iD.at[k]], ssem[b], add=True)

    def outer(t, carry):
        @pl.when(t == 0)
        def _():
            substage(t, isA, idA, isB, idB, isemB, True, False)

        @pl.when(t != 0)
        def _():
            substage(t, isA, idA, isB, idB, isemB, False, False)

        substage(t, isB, idB, isA, idA, isemA, False, True)
        return carry

    lax.fori_loop(0, NOUTER, outer, 0)

    pltpu.make_async_copy(
        rows[(CPS - 1) % 2], acc_sp.at[idB.at[CPS - 1]],
        ssem[(CPS - 1) % 2]).wait()
    plsc.subcore_barrier()
    pltpu.sync_copy(acc_sp.at[pl.ds(s * RPT, RPT)],
                    out_hbm.at[c].at[pl.ds(s * RPT, RPT)])


@jax.jit
def _sc_agg(x, srcdst, zeros_blk):
    return pl.kernel(
        _sc_agg_body,
        out_type=jax.ShapeDtypeStruct((NC, NP, D), jnp.bfloat16),
        mesh=plsc.VectorSubcoreMesh(core_axis_name="c", subcore_axis_name="s"),
        compiler_params=pltpu.CompilerParams(use_tc_tiling_on_sc=False),
        scratch_types=[
            pltpu.MemorySpace.VMEM_SHARED((NP, D), jnp.bfloat16),
            pltpu.MemorySpace.VMEM((CPS, CH), jnp.int32),
            pltpu.MemorySpace.VMEM((CPS, CH), jnp.int32),
            pltpu.MemorySpace.VMEM((CPS, CH), jnp.int32),
            pltpu.MemorySpace.VMEM((CPS, CH), jnp.int32),
            pltpu.MemorySpace.VMEM((CH, D), jnp.bfloat16),
            pltpu.MemorySpace.VMEM((CH, D), jnp.bfloat16),
        ] + [pltpu.SemaphoreType.DMA] * 6,
    )(x, srcdst, zeros_blk)


def _mlp_pool_body(p_ref, seg_ref, w1_ref, b1_ref, w2_ref, b2_ref,
                   sc_ref, sb_ref, y_ref, pool_ref, pacc):
    i = pl.program_id(0)
    h = p_ref[0].astype(jnp.float32) + p_ref[1].astype(jnp.float32)
    h1 = jnp.maximum(
        jnp.dot(h, w1_ref[...], preferred_element_type=jnp.float32)
        + b1_ref[...], 0.0)
    h2 = jnp.maximum(
        jnp.dot(h1, w2_ref[...], preferred_element_type=jnp.float32)
        + b2_ref[...], 0.0)
    y = h2 * sc_ref[...] + sb_ref[...]
    y_ref[...] = y.astype(jnp.bfloat16)

    oh = (lax.broadcasted_iota(jnp.int32, (G, BLK), 0)
          == seg_ref[0]).astype(jnp.float32)

    @pl.when(i == 0)
    def _():
        pacc[...] = jnp.zeros_like(pacc)

    pacc[...] += jnp.dot(oh, y, preferred_element_type=jnp.float32)

    @pl.when(i == pl.num_programs(0) - 1)
    def _():
        pool_ref[...] = pacc[...]


@jax.jit
def _mlp_pool(p, seg, w1, b1, w2, b2, scale, bias):
    return pl.pallas_call(
        _mlp_pool_body,
        grid=(NBLK,),
        in_specs=[
            pl.BlockSpec((NC, BLK, D), lambda i: (0, i, 0)),
            pl.BlockSpec((1, 1, BLK), lambda i: (i, 0, 0)),
            pl.BlockSpec((D, D), lambda i: (0, 0)),
            pl.BlockSpec((1, D), lambda i: (0, 0)),
            pl.BlockSpec((D, D), lambda i: (0, 0)),
            pl.BlockSpec((1, D), lambda i: (0, 0)),
            pl.BlockSpec((1, D), lambda i: (0, 0)),
            pl.BlockSpec((1, D), lambda i: (0, 0)),
        ],
        out_specs=[
            pl.BlockSpec((BLK, D), lambda i: (i, 0)),
            pl.BlockSpec((G, D), lambda i: (0, 0)),
        ],
        out_shape=[
            jax.ShapeDtypeStruct((NP, D), jnp.bfloat16),
            jax.ShapeDtypeStruct((G, D), jnp.float32),
        ],
        scratch_shapes=[pltpu.VMEM((G, D), jnp.float32)],
    )(p, seg, w1, b1, w2, b2, scale, bias)


def _pool_body(x_ref, seg_ref, pool_ref, pacc):
    i = pl.program_id(0)
    oh = (lax.broadcasted_iota(jnp.int32, (G, BLK), 0)
          == seg_ref[0]).astype(jnp.float32)

    @pl.when(i == 0)
    def _():
        pacc[...] = jnp.zeros_like(pacc)

    pacc[...] += jnp.dot(oh, x_ref[...], preferred_element_type=jnp.float32)

    @pl.when(i == pl.num_programs(0) - 1)
    def _():
        pool_ref[...] = pacc[...]


@jax.jit
def _pool(x, seg):
    return pl.pallas_call(
        _pool_body,
        grid=(NBLK,),
        in_specs=[
            pl.BlockSpec((BLK, D), lambda i: (i, 0)),
            pl.BlockSpec((1, 1, BLK), lambda i: (i, 0, 0)),
        ],
        out_specs=pl.BlockSpec((G, D), lambda i: (0, 0)),
        out_shape=jax.ShapeDtypeStruct((G, D), jnp.float32),
        scratch_shapes=[pltpu.VMEM((G, D), jnp.float32)],
    )(x, seg)


def _head_body(pools_ref, fc1w_ref, fc1b_ref, fc1s_ref, fc1t_ref,
               fc2w_ref, fc2b_ref, fc2s_ref, fc2t_ref,
               linw_ref, linb_ref, out_ref):
    def fc(h, w, b, s, t):
        z = jnp.maximum(
            jnp.dot(h, w[...], preferred_element_type=jnp.float32) + b[...],
            0.0)
        return z * s[...] + t[...]

    g = fc(pools_ref[0], fc1w_ref, fc1b_ref, fc1s_ref, fc1t_ref)
    acc = g
    for i in range(1, 6):
        g = fc(g + pools_ref[i], fc2w_ref, fc2b_ref, fc2s_ref, fc2t_ref)
        acc = acc + g
    logits = (jnp.dot(acc, linw_ref[...], preferred_element_type=jnp.float32)
              + linb_ref[...])
    m = jnp.max(logits, axis=-1, keepdims=True)
    z = logits - m
    out_ref[...] = z - jnp.log(jnp.sum(jnp.exp(z), axis=-1, keepdims=True))


@jax.jit
def _head(pools, fc1w, fc1b, fc1s, fc1t, fc2w, fc2b, fc2s, fc2t, linw, linb):
    return pl.pallas_call(
        _head_body,
        out_shape=jax.ShapeDtypeStruct((G, C), jnp.float32),
    )(pools, fc1w, fc1b, fc1s, fc1t, fc2w, fc2b, fc2s, fc2t, linw, linb)


_BN = 1.0 / math.sqrt(1.0 + 1e-5)


def kernel(x, edge_index, batch, params):
    srcdst = edge_index.astype(jnp.int32).reshape(2, NW, NCHUNK, CH)
    seg = jnp.pad(batch.astype(jnp.int32), (0, NP - N),
                  constant_values=G).reshape(NBLK, 1, BLK)
    zeros_blk = jnp.zeros((RPT, D), jnp.bfloat16)
    x = jnp.pad(x, ((0, NP - N), (0, 0)))
    p = params

    def row(v):
        return v.reshape(1, -1)

    pools = [_pool(x, seg)]
    h = x.astype(jnp.bfloat16)
    for c in ["c1", "c2", "c3", "c4", "c5"]:
        parts = _sc_agg(h, srcdst, zeros_blk)
        h, pl_c = _mlp_pool(parts, seg,
                            p[c + "_W1"], row(p[c + "_b1"]),
                            p[c + "_W2"], row(p[c + "_b2"]),
                            row(p[c + "_g"] * _BN), row(p[c + "_bb"]))
        pools.append(pl_c)

    pools = jnp.stack(pools)
    return _head(pools,
                 p["fc1_W"], row(p["fc1_b"]), row(p["fc1_g"] * _BN),
                 row(p["fc1_bb"]),
                 p["fc2_W"], row(p["fc2_b"]), row(p["fc2_g"] * _BN),
                 row(p["fc2_bb"]),
                 p["lin_W"], row(p["lin_b"]))
